# Initial kernel scaffold; baseline (speedup 1.0000x reference)
import jax
import jax.numpy as jnp
from jax.experimental import pallas as pl


def kernel(x, edge_index, W1, as1, ad1, b1, W2, as2, ad2, b2, W3, as3, ad3, b3, W4, as4, ad4, b4):
    raise NotImplementedError("write your pallas kernel here")


# speedup vs baseline: 9.3787x; 9.3787x over previous
import functools

import jax
import jax.numpy as jnp
from jax import lax
from jax.experimental import pallas as pl
from jax.experimental.pallas import tpu as pltpu
from jax.experimental.pallas import tpu_sc as plsc

N = 10000
D = 256
HDIM = 128
E_RAW = 160000
E1 = E_RAW + N
NC = 2
NS = 16
LANES = 16
E_PAD = 172032
K2 = E_PAD // NS
CH = 128
G = K2 // CH
NP = 10240
STRIPE = NP // NS
FB = 128
WC = 64
JV = WC // LANES
LEAK = 0.2


def _tc_body(relu_in, nin, nout, x_ref, w_ref, att_ref, h_ref, aa_ref):
    xs = []
    for i in range(nin):
        xi = x_ref[i]
        if relu_in:
            xi = jnp.maximum(xi, 0.0)
        xs.append(xi)
    aa = None
    for q in range(nout):
        hq = jnp.dot(xs[0], w_ref[0, q], preferred_element_type=jnp.float32)
        for i in range(1, nin):
            hq = hq + jnp.dot(xs[i], w_ref[i, q],
                              preferred_element_type=jnp.float32)
        h_ref[q] = hq
        aq = jnp.dot(hq, att_ref[q], preferred_element_type=jnp.float32)
        aa = aq if aa is None else aa + aq
    s = aa[:, 0:1] + aa[:, 1:2]
    c = jnp.where(s > 0, s, LEAK * s)
    aa_ref[...] = jnp.concatenate(
        [aa[:, 0:2], c, jnp.zeros((aa.shape[0], 5), jnp.float32)], axis=1)


@functools.partial(jax.jit, static_argnums=(3, 4, 5))
def _tc_layer(x_all, w_all, att_r, nin, nout, relu_in):
    B = 1000
    grid = (N // B,)
    return pl.pallas_call(
        functools.partial(_tc_body, relu_in, nin, nout),
        grid=grid,
        in_specs=[
            pl.BlockSpec((nin, B, WC), lambda i: (0, i, 0)),
            pl.BlockSpec((nin, nout, WC, WC), lambda i: (0, 0, 0, 0)),
            pl.BlockSpec((nout, WC, 8), lambda i: (0, 0, 0)),
        ],
        out_specs=[
            pl.BlockSpec((nout, B, WC), lambda i: (0, i, 0)),
            pl.BlockSpec((B, 8), lambda i: (i, 0)),
        ],
        out_shape=[
            jax.ShapeDtypeStruct((nout, N, WC), jnp.float32),
            jax.ShapeDtypeStruct((N, 8), jnp.float32),
        ],
    )(x_all, w_all, att_r)


def _sc_body(nout, h_hbm, as_hbm, ad_hbm, src_hbm, dst_hbm, b_hbm,
             out_hbm,
             asn, adn, denomn, srcv, dstv, wv, sidx, didx, rows,
             biasv, dacc, dtmp, partials, sdenom, out_acc, sem):
    qc = nout // NC
    cid = lax.axis_index("c")
    sid = lax.axis_index("s")
    base = sid * K2
    nvec16 = K2 // LANES
    zero16 = jnp.zeros((LANES,), jnp.float32)
    soff = sid * STRIPE

    pltpu.sync_copy(as_hbm, asn)
    pltpu.sync_copy(ad_hbm, adn)
    pltpu.sync_copy(src_hbm.at[pl.ds(base, K2)], srcv)
    pltpu.sync_copy(dst_hbm.at[pl.ds(base, K2)], dstv)

    def _zden(i, _):
        denomn[pl.ds(i * LANES, LANES)] = zero16
        return 0
    lax.fori_loop(0, NP // LANES, _zden, 0)

    iota16 = lax.iota(jnp.int32, 16)

    def _p1(v, _):
        off = v * LANES
        s16 = srcv[pl.ds(off, LANES)]
        d16 = dstv[pl.ds(off, LANES)]
        asg = plsc.load_gather(asn, [s16])
        adg = plsc.load_gather(adn, [d16])
        e = asg + adg
        e = jnp.where(e > 0, e, LEAK * e)
        sld = plsc.load_gather(asn, [d16]) + adg
        cg = jnp.where(sld > 0, sld, LEAK * sld)
        w = jnp.exp(e - cg)
        eid = base + off + iota16
        w = jnp.where(eid < E1, w, 0.0)
        wv[pl.ds(off, LANES)] = w
        plsc.addupdate_scatter(denomn, [d16], w)
        return 0
    lax.fori_loop(0, nvec16, _p1, 0)

    pltpu.sync_copy(denomn, partials.at[sid])
    plsc.subcore_barrier()

    def _zacc(i, _):
        dacc[pl.ds(i * LANES, LANES)] = zero16
        return 0
    lax.fori_loop(0, STRIPE // LANES, _zacc, 0)
    for t in range(NS):
        pltpu.sync_copy(partials.at[t, pl.ds(soff, STRIPE)], dtmp)

        def _acc(i, _):
            sl = pl.ds(i * LANES, LANES)
            dacc[sl] = dacc[sl] + dtmp[sl]
            return 0
        lax.fori_loop(0, STRIPE // LANES, _acc, 0)
    pltpu.sync_copy(dacc, sdenom.at[pl.ds(soff, STRIPE)])
    plsc.subcore_barrier()
    pltpu.sync_copy(sdenom, denomn)

    def _pa(v, _):
        off = v * LANES
        d16 = dstv[pl.ds(off, LANES)]
        den = plsc.load_gather(denomn, [d16])
        wv[pl.ds(off, LANES)] = wv[pl.ds(off, LANES)] / (den + 1e-16)
        return 0
    lax.fori_loop(0, nvec16, _pa, 0)

    for qq in range(qc):
        q = cid * qc + qq

        def _zrows(i, _):
            for j in range(JV):
                rows[i, pl.ds(j * LANES, LANES)] = zero16
            return 0
        lax.fori_loop(0, FB, _zrows, 0)
        for j in range(STRIPE // FB):
            pltpu.sync_copy(rows, out_acc.at[pl.ds(soff + j * FB, FB)])
        plsc.subcore_barrier()

        def _p3(g, _):
            goff = g * CH
            for v in range(CH // LANES):
                off = goff + v * LANES
                didx[pl.ds(v * LANES, LANES)] = dstv[pl.ds(off, LANES)]
                sidx[pl.ds(v * LANES, LANES)] = srcv[pl.ds(off, LANES)]
            pltpu.async_copy(h_hbm.at[q].at[sidx], rows, sem).wait()

            def _scale(k, _):
                al16 = plsc.load_gather(
                    wv, [jnp.full((LANES,), goff + k, jnp.int32)])
                for j in range(JV):
                    sl = pl.ds(j * LANES, LANES)
                    rows[k, sl] = rows[k, sl] * al16
                return 0
            lax.fori_loop(0, CH, _scale, 0)
            pltpu.sync_copy(rows, out_acc.at[didx], add=True)
            return 0
        lax.fori_loop(0, G, _p3, 0)

        plsc.subcore_barrier()
        pltpu.sync_copy(b_hbm.at[q], biasv)
        for j in range(STRIPE // FB):
            rstart = soff + j * FB

            @pl.when(rstart + FB <= N)
            def _full():
                pltpu.sync_copy(out_acc.at[pl.ds(rstart, FB)], rows)

                def _badd(r, _):
                    for jj in range(JV):
                        sl = pl.ds(jj * LANES, LANES)
                        rows[r, sl] = rows[r, sl] + biasv[sl]
                    return 0
                lax.fori_loop(0, FB, _badd, 0)
                pltpu.sync_copy(rows, out_hbm.at[q].at[pl.ds(rstart, FB)])

            nrem = N % FB
            if nrem:
                @pl.when(jnp.logical_and(rstart < N, rstart + FB > N))
                def _part():
                    pltpu.sync_copy(out_acc.at[pl.ds(rstart, nrem)],
                                    rows.at[pl.ds(0, nrem)])

                    def _badd(r, _):
                        for jj in range(JV):
                            sl = pl.ds(jj * LANES, LANES)
                            rows[r, sl] = rows[r, sl] + biasv[sl]
                        return 0
                    lax.fori_loop(0, nrem, _badd, 0)
                    pltpu.sync_copy(rows.at[pl.ds(0, nrem)],
                                    out_hbm.at[q].at[pl.ds(rstart, nrem)])
        if qq + 1 < qc:
            plsc.subcore_barrier()


@functools.partial(jax.jit, static_argnums=(6,))
def _sc_layer(h_all, asv, adv, src, dst, b_all, nout):
    mesh = plsc.VectorSubcoreMesh(core_axis_name="c", subcore_axis_name="s",
                                  num_cores=NC, num_subcores=NS)
    kern = pl.kernel(
        functools.partial(_sc_body, nout),
        out_type=jax.ShapeDtypeStruct((nout, N, WC), jnp.float32),
        mesh=mesh,
        compiler_params=pltpu.CompilerParams(needs_layout_passes=False,
                                             use_tc_tiling_on_sc=False),
        scratch_types=[
            pltpu.VMEM((N,), jnp.float32),
            pltpu.VMEM((N,), jnp.float32),
            pltpu.VMEM((NP,), jnp.float32),
            pltpu.VMEM((K2,), jnp.int32),
            pltpu.VMEM((K2,), jnp.int32),
            pltpu.VMEM((K2,), jnp.float32),
            pltpu.VMEM((CH,), jnp.int32),
            pltpu.VMEM((CH,), jnp.int32),
            pltpu.VMEM((CH, WC), jnp.float32),
            pltpu.VMEM((WC,), jnp.float32),
            pltpu.VMEM((STRIPE,), jnp.float32),
            pltpu.VMEM((STRIPE,), jnp.float32),
            pltpu.VMEM_SHARED((NS, NP), jnp.float32),
            pltpu.VMEM_SHARED((NP,), jnp.float32),
            pltpu.VMEM_SHARED((NP, WC), jnp.float32),
            pltpu.SemaphoreType.DMA,
        ],
    )
    return kern(h_all, asv, adv, src, dst, b_all)


def kernel(x, edge_index, W1, as1, ad1, b1, W2, as2, ad2, b2,
           W3, as3, ad3, b3, W4, as4, ad4, b4):
    ar = jnp.arange(N, dtype=jnp.int32)
    pad = jnp.zeros((E_PAD - E1,), jnp.int32)
    src = jnp.concatenate([edge_index[0].astype(jnp.int32), ar, pad])
    dst = jnp.concatenate([edge_index[1].astype(jnp.int32), ar, pad])

    layers = [
        (W1, as1, ad1, b1, D, D, False),
        (W2, as2, ad2, b2, D, HDIM, True),
        (W3, as3, ad3, b3, HDIM, D, True),
        (W4, as4, ad4, b4, D, D, True),
    ]
    x_all = x.reshape(N, D // WC, WC).transpose(1, 0, 2)
    outs = []
    for (W, att_s, att_d, b, di, do, relu_in) in layers:
        nin, nout = di // WC, do // WC
        w_all = W.reshape(nin, WC, nout, WC).transpose(0, 2, 1, 3)
        att8 = jnp.zeros((do, 8), jnp.float32)
        att8 = att8.at[:, 0].set(att_s).at[:, 1].set(att_d)
        att_r = att8.reshape(nout, WC, 8)
        h_all, aa = _tc_layer(x_all, w_all, att_r, nin, nout, relu_in)
        o_all = _sc_layer(h_all, aa[:, 0], aa[:, 1],
                          src, dst, b.reshape(nout, WC), nout)
        outs.append(o_all)
        x_all = o_all

    x4 = outs[3].transpose(1, 0, 2).reshape(N, D)
    h_mid = outs[1].transpose(1, 0, 2).reshape(N, HDIM)
    return (x4, h_mid)


# speedup vs baseline: 12.9045x; 1.3759x over previous
import functools

import jax
import jax.numpy as jnp
from jax import lax
from jax.experimental import pallas as pl
from jax.experimental.pallas import tpu as pltpu
from jax.experimental.pallas import tpu_sc as plsc

N = 10000
D = 256
HDIM = 128
E_RAW = 160000
E1 = E_RAW + N
NC = 2
NS = 16
LANES = 16
E_PAD = 172032
K2 = E_PAD // NS
CH = 112
G = K2 // CH
NP = 10240
STRIPE = NP // NS
FB = 80
WC = 64
JV = WC // LANES
LEAK = 0.2


def _tc_body(relu_in, nin, nout, x_ref, w_ref, att_ref, h_ref, aa_ref):
    xs = []
    for i in range(nin):
        xi = x_ref[i]
        if relu_in:
            xi = jnp.maximum(xi, 0.0)
        xs.append(xi)
    aa = None
    for q in range(nout):
        hq = jnp.dot(xs[0], w_ref[0, q], preferred_element_type=jnp.float32)
        for i in range(1, nin):
            hq = hq + jnp.dot(xs[i], w_ref[i, q],
                              preferred_element_type=jnp.float32)
        h_ref[q] = hq
        aq = jnp.dot(hq, att_ref[q], preferred_element_type=jnp.float32)
        aa = aq if aa is None else aa + aq
    s = aa[:, 0:1] + aa[:, 1:2]
    c = jnp.where(s > 0, s, LEAK * s)
    aa_ref[...] = jnp.concatenate(
        [aa[:, 0:2], c, jnp.zeros((aa.shape[0], 5), jnp.float32)], axis=1)


@functools.partial(jax.jit, static_argnums=(3, 4, 5))
def _tc_layer(x_all, w_all, att_r, nin, nout, relu_in):
    B = 1000
    grid = (N // B,)
    return pl.pallas_call(
        functools.partial(_tc_body, relu_in, nin, nout),
        grid=grid,
        in_specs=[
            pl.BlockSpec((nin, B, WC), lambda i: (0, i, 0)),
            pl.BlockSpec((nin, nout, WC, WC), lambda i: (0, 0, 0, 0)),
            pl.BlockSpec((nout, WC, 8), lambda i: (0, 0, 0)),
        ],
        out_specs=[
            pl.BlockSpec((nout, B, WC), lambda i: (0, i, 0)),
            pl.BlockSpec((B, 8), lambda i: (i, 0)),
        ],
        out_shape=[
            jax.ShapeDtypeStruct((nout, N, WC), jnp.float32),
            jax.ShapeDtypeStruct((N, 8), jnp.float32),
        ],
    )(x_all, w_all, att_r)


def _sc_body(nout, h_hbm, as_hbm, ad_hbm, src_hbm, dst_hbm, b_hbm,
             out_hbm,
             asn, adn, denomn, srcv, dstv, wv, didx0, didx1, rows0, rows1,
             biasv, dacc, dtmp, partials, sdenom, out_acc,
             gsem0, gsem1, ssem0, ssem1):
    didx = (didx0, didx1)
    rows = (rows0, rows1)
    gsem = (gsem0, gsem1)
    ssem = (ssem0, ssem1)
    qc = nout // NC
    cid = lax.axis_index("c")
    sid = lax.axis_index("s")
    base = sid * K2
    nvec16 = K2 // LANES
    zero16 = jnp.zeros((LANES,), jnp.float32)
    soff = sid * STRIPE

    pltpu.sync_copy(as_hbm, asn)
    pltpu.sync_copy(ad_hbm, adn)
    pltpu.sync_copy(src_hbm.at[pl.ds(base, K2)], srcv)
    pltpu.sync_copy(dst_hbm.at[pl.ds(base, K2)], dstv)

    def _zden(i, _):
        denomn[pl.ds(i * LANES, LANES)] = zero16
        return 0
    lax.fori_loop(0, NP // LANES, _zden, 0)

    iota16 = lax.iota(jnp.int32, 16)

    def _p1(v, _):
        off = v * LANES
        s16 = srcv[pl.ds(off, LANES)]
        d16 = dstv[pl.ds(off, LANES)]
        asg = plsc.load_gather(asn, [s16])
        adg = plsc.load_gather(adn, [d16])
        e = asg + adg
        e = jnp.where(e > 0, e, LEAK * e)
        sld = plsc.load_gather(asn, [d16]) + adg
        cg = jnp.where(sld > 0, sld, LEAK * sld)
        w = jnp.exp(e - cg)
        eid = base + off + iota16
        w = jnp.where(eid < E1, w, 0.0)
        wv[pl.ds(off, LANES)] = w
        plsc.addupdate_scatter(denomn, [d16], w)
        return 0
    lax.fori_loop(0, nvec16, _p1, 0, unroll=4)

    pltpu.sync_copy(denomn, partials.at[sid])
    plsc.subcore_barrier()

    def _zacc(i, _):
        dacc[pl.ds(i * LANES, LANES)] = zero16
        return 0
    lax.fori_loop(0, STRIPE // LANES, _zacc, 0)
    for t in range(NS):
        pltpu.sync_copy(partials.at[t, pl.ds(soff, STRIPE)], dtmp)

        def _acc(i, _):
            sl = pl.ds(i * LANES, LANES)
            dacc[sl] = dacc[sl] + dtmp[sl]
            return 0
        lax.fori_loop(0, STRIPE // LANES, _acc, 0)
    pltpu.sync_copy(dacc, sdenom.at[pl.ds(soff, STRIPE)])
    plsc.subcore_barrier()
    pltpu.sync_copy(sdenom, denomn)

    def _pa(v, _):
        off = v * LANES
        d16 = dstv[pl.ds(off, LANES)]
        den = plsc.load_gather(denomn, [d16])
        wv[pl.ds(off, LANES)] = wv[pl.ds(off, LANES)] / (den + 1e-16)
        return 0
    lax.fori_loop(0, nvec16, _pa, 0, unroll=4)

    def _gidx(g):
        return srcv.at[pl.ds(g * CH, CH)]

    for qq in range(qc):
        q = cid * qc + qq

        def _zrows(i, _):
            for j in range(JV):
                rows0[i, pl.ds(j * LANES, LANES)] = zero16
            return 0
        lax.fori_loop(0, FB, _zrows, 0)
        for j in range(STRIPE // FB):
            pltpu.sync_copy(rows0.at[pl.ds(0, FB)],
                            out_acc.at[pl.ds(soff + j * FB, FB)])
        plsc.subcore_barrier()

        pltpu.async_copy(h_hbm.at[q].at[_gidx(0)], rows0, gsem0)

        def _pair(p, _):
            for b in range(2):
                g = p * 2 + b
                nb = 1 - b
                gn = g + 1

                @pl.when(gn < G)
                def _pre():
                    @pl.when(gn >= 2)
                    def _drain():
                        pltpu.make_async_copy(
                            rows[nb], out_acc.at[didx[nb]], ssem[nb]).wait()
                    pltpu.async_copy(h_hbm.at[q].at[_gidx(gn)],
                                     rows[nb], gsem[nb])

                pltpu.make_async_copy(h_hbm.at[q].at[_gidx(g)],
                                      rows[b], gsem[b]).wait()

                goff = g * CH
                for v in range(CH // LANES):
                    didx[b][pl.ds(v * LANES, LANES)] = (
                        dstv[pl.ds(goff + v * LANES, LANES)])

                def _scale(k, _):
                    al16 = plsc.load_gather(
                        wv, [jnp.full((LANES,), goff + k, jnp.int32)])
                    for j in range(JV):
                        sl = pl.ds(j * LANES, LANES)
                        rows[b][k, sl] = rows[b][k, sl] * al16
                    return 0
                lax.fori_loop(0, CH, _scale, 0, unroll=8)
                pltpu.async_copy(rows[b], out_acc.at[didx[b]], ssem[b],
                                 add=True)
            return 0
        lax.fori_loop(0, G // 2, _pair, 0)
        pltpu.make_async_copy(rows0, out_acc.at[didx0], ssem0).wait()
        pltpu.make_async_copy(rows1, out_acc.at[didx1], ssem1).wait()

        plsc.subcore_barrier()
        pltpu.sync_copy(b_hbm.at[q], biasv)
        for j in range(STRIPE // FB):
            rstart = soff + j * FB

            @pl.when(rstart + FB <= N)
            def _full():
                pltpu.sync_copy(out_acc.at[pl.ds(rstart, FB)],
                                rows0.at[pl.ds(0, FB)])

                def _badd(r, _):
                    for jj in range(JV):
                        sl = pl.ds(jj * LANES, LANES)
                        rows0[r, sl] = rows0[r, sl] + biasv[sl]
                    return 0
                lax.fori_loop(0, FB, _badd, 0, unroll=8)
                pltpu.sync_copy(rows0.at[pl.ds(0, FB)],
                                out_hbm.at[q].at[pl.ds(rstart, FB)])
        if qq + 1 < qc:
            plsc.subcore_barrier()


@functools.partial(jax.jit, static_argnums=(6,))
def _sc_layer(h_all, asv, adv, src, dst, b_all, nout):
    mesh = plsc.VectorSubcoreMesh(core_axis_name="c", subcore_axis_name="s",
                                  num_cores=NC, num_subcores=NS)
    kern = pl.kernel(
        functools.partial(_sc_body, nout),
        out_type=jax.ShapeDtypeStruct((nout, N, WC), jnp.float32),
        mesh=mesh,
        compiler_params=pltpu.CompilerParams(needs_layout_passes=False,
                                             use_tc_tiling_on_sc=False),
        scratch_types=[
            pltpu.VMEM((N,), jnp.float32),
            pltpu.VMEM((N,), jnp.float32),
            pltpu.VMEM((NP,), jnp.float32),
            pltpu.VMEM((K2,), jnp.int32),
            pltpu.VMEM((K2,), jnp.int32),
            pltpu.VMEM((K2,), jnp.float32),
            pltpu.VMEM((CH,), jnp.int32),
            pltpu.VMEM((CH,), jnp.int32),
            pltpu.VMEM((CH, WC), jnp.float32),
            pltpu.VMEM((CH, WC), jnp.float32),
            pltpu.VMEM((WC,), jnp.float32),
            pltpu.VMEM((STRIPE,), jnp.float32),
            pltpu.VMEM((STRIPE,), jnp.float32),
            pltpu.VMEM_SHARED((NS, NP), jnp.float32),
            pltpu.VMEM_SHARED((NP,), jnp.float32),
            pltpu.VMEM_SHARED((NP, WC), jnp.float32),
            pltpu.SemaphoreType.DMA,
            pltpu.SemaphoreType.DMA,
            pltpu.SemaphoreType.DMA,
            pltpu.SemaphoreType.DMA,
        ],
    )
    return kern(h_all, asv, adv, src, dst, b_all)


def kernel(x, edge_index, W1, as1, ad1, b1, W2, as2, ad2, b2,
           W3, as3, ad3, b3, W4, as4, ad4, b4):
    ar = jnp.arange(N, dtype=jnp.int32)
    pad = jnp.zeros((E_PAD - E1,), jnp.int32)
    src = jnp.concatenate([edge_index[0].astype(jnp.int32), ar, pad])
    dst = jnp.concatenate([edge_index[1].astype(jnp.int32), ar, pad])

    layers = [
        (W1, as1, ad1, b1, D, D, False),
        (W2, as2, ad2, b2, D, HDIM, True),
        (W3, as3, ad3, b3, HDIM, D, True),
        (W4, as4, ad4, b4, D, D, True),
    ]
    x_all = x.reshape(N, D // WC, WC).transpose(1, 0, 2)
    outs = []
    for (W, att_s, att_d, b, di, do, relu_in) in layers:
        nin, nout = di // WC, do // WC
        w_all = W.reshape(nin, WC, nout, WC).transpose(0, 2, 1, 3)
        att8 = jnp.zeros((do, 8), jnp.float32)
        att8 = att8.at[:, 0].set(att_s).at[:, 1].set(att_d)
        att_r = att8.reshape(nout, WC, 8)
        h_all, aa = _tc_layer(x_all, w_all, att_r, nin, nout, relu_in)
        o_all = _sc_layer(h_all, aa[:, 0], aa[:, 1],
                          src, dst, b.reshape(nout, WC), nout)
        outs.append(o_all)
        x_all = o_all

    x4 = outs[3].transpose(1, 0, 2).reshape(N, D)
    h_mid = outs[1].transpose(1, 0, 2).reshape(N, HDIM)
    return (x4, h_mid)


# speedup vs baseline: 15.5118x; 1.2021x over previous
import functools

import jax
import jax.numpy as jnp
from jax import lax
from jax.experimental import pallas as pl
from jax.experimental.pallas import tpu as pltpu
from jax.experimental.pallas import tpu_sc as plsc

N = 10000
D = 256
HDIM = 128
E_RAW = 160000
E1 = E_RAW + N
NC = 2
NS = 16
LANES = 16
E_PAD = 172032
K2 = E_PAD // NS
CH = 112
G = K2 // CH
NP = 10240
STRIPE = NP // NS
FB = 80
WC = 64
JV = WC // LANES
LEAK = 0.2


def _tc_body(relu_in, nin, nout, x_ref, w_ref, att_ref, h_ref, aa_ref):
    xs = []
    for i in range(nin):
        xi = x_ref[i]
        if relu_in:
            xi = jnp.maximum(xi, 0.0)
        xs.append(xi)
    aa = None
    for q in range(nout):
        hq = jnp.dot(xs[0], w_ref[0, q], preferred_element_type=jnp.float32)
        for i in range(1, nin):
            hq = hq + jnp.dot(xs[i], w_ref[i, q],
                              preferred_element_type=jnp.float32)
        h_ref[q] = hq
        aq = jnp.dot(hq, att_ref[q], preferred_element_type=jnp.float32)
        aa = aq if aa is None else aa + aq
    s = aa[:, 0:1] + aa[:, 1:2]
    c = jnp.where(s > 0, s, LEAK * s)
    aa_ref[...] = jnp.concatenate(
        [aa[:, 0:2], c, jnp.zeros((aa.shape[0], 5), jnp.float32)], axis=1)


@functools.partial(jax.jit, static_argnums=(3, 4, 5))
def _tc_layer(x_all, w_all, att_r, nin, nout, relu_in):
    B = 1000
    grid = (N // B,)
    return pl.pallas_call(
        functools.partial(_tc_body, relu_in, nin, nout),
        grid=grid,
        in_specs=[
            pl.BlockSpec((nin, B, WC), lambda i: (0, i, 0)),
            pl.BlockSpec((nin, nout, WC, WC), lambda i: (0, 0, 0, 0)),
            pl.BlockSpec((nout, WC, 8), lambda i: (0, 0, 0)),
        ],
        out_specs=[
            pl.BlockSpec((nout, B, WC), lambda i: (0, i, 0)),
            pl.BlockSpec((B, 8), lambda i: (i, 0)),
        ],
        out_shape=[
            jax.ShapeDtypeStruct((nout, N, WC), jnp.float32),
            jax.ShapeDtypeStruct((N, 8), jnp.float32),
        ],
    )(x_all, w_all, att_r)


def _sc_body(nout, h_hbm, as_hbm, ad_hbm, src_hbm, dst_hbm, b_hbm,
             out_hbm,
             asn, adn, denomn, srcv, dstv, wv, didx0, didx1, rows0, rows1,
             biasv, dacc, dtmp, partials, sdenom, out_acc,
             gsem0, gsem1, ssem0, ssem1):
    didx = (didx0, didx1)
    rows = (rows0, rows1)
    gsem = (gsem0, gsem1)
    ssem = (ssem0, ssem1)
    qc = nout // NC
    cid = lax.axis_index("c")
    sid = lax.axis_index("s")
    base = sid * K2
    nvec16 = K2 // LANES
    zero16 = jnp.zeros((LANES,), jnp.float32)
    soff = sid * STRIPE

    pltpu.sync_copy(as_hbm, asn)
    pltpu.sync_copy(ad_hbm, adn)
    pltpu.sync_copy(src_hbm.at[pl.ds(base, K2)], srcv)
    pltpu.sync_copy(dst_hbm.at[pl.ds(base, K2)], dstv)

    def _zden(i, _):
        denomn[pl.ds(i * LANES, LANES)] = zero16
        return 0
    lax.fori_loop(0, NP // LANES, _zden, 0)

    iota16 = lax.iota(jnp.int32, 16)

    def _p1(v, _):
        off = v * LANES
        s16 = srcv[pl.ds(off, LANES)]
        d16 = dstv[pl.ds(off, LANES)]
        asg = plsc.load_gather(asn, [s16])
        adg = plsc.load_gather(adn, [d16])
        e = asg + adg
        e = jnp.where(e > 0, e, LEAK * e)
        sld = plsc.load_gather(asn, [d16]) + adg
        cg = jnp.where(sld > 0, sld, LEAK * sld)
        w = jnp.exp(e - cg)
        eid = base + off + iota16
        w = jnp.where(eid < E1, w, 0.0)
        wv[pl.ds(off, LANES)] = w
        plsc.addupdate_scatter(denomn, [d16], w)
        return 0
    lax.fori_loop(0, nvec16, _p1, 0, unroll=4)

    pltpu.sync_copy(denomn, partials.at[sid])
    plsc.subcore_barrier()

    def _zacc(i, _):
        dacc[pl.ds(i * LANES, LANES)] = zero16
        return 0
    lax.fori_loop(0, STRIPE // LANES, _zacc, 0)
    for t in range(NS):
        pltpu.sync_copy(partials.at[t, pl.ds(soff, STRIPE)], dtmp)

        def _acc(i, _):
            sl = pl.ds(i * LANES, LANES)
            dacc[sl] = dacc[sl] + dtmp[sl]
            return 0
        lax.fori_loop(0, STRIPE // LANES, _acc, 0)
    pltpu.sync_copy(dacc, sdenom.at[pl.ds(soff, STRIPE)])
    plsc.subcore_barrier()
    pltpu.sync_copy(sdenom, denomn)

    def _pa(v, _):
        off = v * LANES
        d16 = dstv[pl.ds(off, LANES)]
        den = plsc.load_gather(denomn, [d16])
        wv[pl.ds(off, LANES)] = wv[pl.ds(off, LANES)] / (den + 1e-16)
        return 0
    lax.fori_loop(0, nvec16, _pa, 0, unroll=4)

    def _gidx(g):
        return srcv.at[pl.ds(g * CH, CH)]

    for qq in range(qc):
        q = cid * qc + qq

        def _zrows(i, _):
            for j in range(JV):
                rows0[i, pl.ds(j * LANES, LANES)] = zero16
            return 0
        lax.fori_loop(0, FB, _zrows, 0)
        for j in range(STRIPE // FB):
            pltpu.sync_copy(rows0.at[pl.ds(0, FB)],
                            out_acc.at[pl.ds(soff + j * FB, FB)])
        plsc.subcore_barrier()

        pltpu.async_copy(h_hbm.at[q].at[_gidx(0)], rows0, gsem0)

        def _pair(p, _):
            for b in range(2):
                g = p * 2 + b
                nb = 1 - b
                gn = g + 1

                @pl.when(gn < G)
                def _pre():
                    @pl.when(gn >= 2)
                    def _drain():
                        pltpu.make_async_copy(
                            rows[nb], out_acc.at[didx[nb]], ssem[nb]).wait()
                    pltpu.async_copy(h_hbm.at[q].at[_gidx(gn)],
                                     rows[nb], gsem[nb])

                pltpu.make_async_copy(h_hbm.at[q].at[_gidx(g)],
                                      rows[b], gsem[b]).wait()

                goff = g * CH
                for v in range(CH // LANES):
                    didx[b][pl.ds(v * LANES, LANES)] = (
                        dstv[pl.ds(goff + v * LANES, LANES)])

                def _scale(k, _):
                    al16 = plsc.load_gather(
                        wv, [jnp.full((LANES,), goff + k, jnp.int32)])
                    for j in range(JV):
                        sl = pl.ds(j * LANES, LANES)
                        rows[b][k, sl] = rows[b][k, sl] * al16
                    return 0
                lax.fori_loop(0, 1, _scale, 0, unroll=8)
                pltpu.async_copy(rows[b], out_acc.at[didx[b]], ssem[b],
                                 add=True)
            return 0
        lax.fori_loop(0, G // 2, _pair, 0)
        pltpu.make_async_copy(rows0, out_acc.at[didx0], ssem0).wait()
        pltpu.make_async_copy(rows1, out_acc.at[didx1], ssem1).wait()

        plsc.subcore_barrier()
        pltpu.sync_copy(b_hbm.at[q], biasv)
        for j in range(STRIPE // FB):
            rstart = soff + j * FB

            @pl.when(rstart + FB <= N)
            def _full():
                pltpu.sync_copy(out_acc.at[pl.ds(rstart, FB)],
                                rows0.at[pl.ds(0, FB)])

                def _badd(r, _):
                    for jj in range(JV):
                        sl = pl.ds(jj * LANES, LANES)
                        rows0[r, sl] = rows0[r, sl] + biasv[sl]
                    return 0
                lax.fori_loop(0, FB, _badd, 0, unroll=8)
                pltpu.sync_copy(rows0.at[pl.ds(0, FB)],
                                out_hbm.at[q].at[pl.ds(rstart, FB)])
        if qq + 1 < qc:
            plsc.subcore_barrier()


@functools.partial(jax.jit, static_argnums=(6,))
def _sc_layer(h_all, asv, adv, src, dst, b_all, nout):
    mesh = plsc.VectorSubcoreMesh(core_axis_name="c", subcore_axis_name="s",
                                  num_cores=NC, num_subcores=NS)
    kern = pl.kernel(
        functools.partial(_sc_body, nout),
        out_type=jax.ShapeDtypeStruct((nout, N, WC), jnp.float32),
        mesh=mesh,
        compiler_params=pltpu.CompilerParams(needs_layout_passes=False,
                                             use_tc_tiling_on_sc=False),
        scratch_types=[
            pltpu.VMEM((N,), jnp.float32),
            pltpu.VMEM((N,), jnp.float32),
            pltpu.VMEM((NP,), jnp.float32),
            pltpu.VMEM((K2,), jnp.int32),
            pltpu.VMEM((K2,), jnp.int32),
            pltpu.VMEM((K2,), jnp.float32),
            pltpu.VMEM((CH,), jnp.int32),
            pltpu.VMEM((CH,), jnp.int32),
            pltpu.VMEM((CH, WC), jnp.float32),
            pltpu.VMEM((CH, WC), jnp.float32),
            pltpu.VMEM((WC,), jnp.float32),
            pltpu.VMEM((STRIPE,), jnp.float32),
            pltpu.VMEM((STRIPE,), jnp.float32),
            pltpu.VMEM_SHARED((NS, NP), jnp.float32),
            pltpu.VMEM_SHARED((NP,), jnp.float32),
            pltpu.VMEM_SHARED((NP, WC), jnp.float32),
            pltpu.SemaphoreType.DMA,
            pltpu.SemaphoreType.DMA,
            pltpu.SemaphoreType.DMA,
            pltpu.SemaphoreType.DMA,
        ],
    )
    return kern(h_all, asv, adv, src, dst, b_all)


def kernel(x, edge_index, W1, as1, ad1, b1, W2, as2, ad2, b2,
           W3, as3, ad3, b3, W4, as4, ad4, b4):
    ar = jnp.arange(N, dtype=jnp.int32)
    pad = jnp.zeros((E_PAD - E1,), jnp.int32)
    src = jnp.concatenate([edge_index[0].astype(jnp.int32), ar, pad])
    dst = jnp.concatenate([edge_index[1].astype(jnp.int32), ar, pad])

    layers = [
        (W1, as1, ad1, b1, D, D, False),
        (W2, as2, ad2, b2, D, HDIM, True),
        (W3, as3, ad3, b3, HDIM, D, True),
        (W4, as4, ad4, b4, D, D, True),
    ]
    x_all = x.reshape(N, D // WC, WC).transpose(1, 0, 2)
    outs = []
    for (W, att_s, att_d, b, di, do, relu_in) in layers:
        nin, nout = di // WC, do // WC
        w_all = W.reshape(nin, WC, nout, WC).transpose(0, 2, 1, 3)
        att8 = jnp.zeros((do, 8), jnp.float32)
        att8 = att8.at[:, 0].set(att_s).at[:, 1].set(att_d)
        att_r = att8.reshape(nout, WC, 8)
        h_all, aa = _tc_layer(x_all, w_all, att_r, nin, nout, relu_in)
        o_all = _sc_layer(h_all, aa[:, 0], aa[:, 1],
                          src, dst, b.reshape(nout, WC), nout)
        outs.append(o_all)
        x_all = o_all

    x4 = outs[3].transpose(1, 0, 2).reshape(N, D)
    h_mid = outs[1].transpose(1, 0, 2).reshape(N, HDIM)
    return (x4, h_mid)
